# baseline (speedup 1.0000x reference)
import functools

import jax
import jax.numpy as jnp
from jax import lax
from jax.experimental import pallas as pl
from jax.experimental.pallas import tpu as pltpu
from jax.experimental.pallas import tpu_sc as plsc

N_ELEM = 119
N_RES = 21
EMB = 128
HID = 384

A_CHUNK = 17
GRID_A = 7
BC_STRIDE = 8
ROWS_PER_STEP = A_CHUNK * N_RES * BC_STRIDE
N_TABLE = GRID_A * ROWS_PER_STEP

NW = 32
B_PER_W = 3200
CHUNK = 64
N_CHUNKS = B_PER_W // CHUNK
LANES = 16
SLOTS = 4

N_ATOMS = 100000
LAST_FULL_LOCAL = N_ATOMS // CHUNK - (NW - 1) * N_CHUNKS
PART = N_ATOMS % CHUNK
assert PART % LANES == 0 and LAST_FULL_LOCAL >= SLOTS


def _mm_t(x, w):
    return lax.dot_general(x, w, (((1,), (1,)), ((), ())),
                           preferred_element_type=jnp.float32)


def _table_body(ae_ref, re_ref, bw_ref, bb_ref, w1_ref, b1_ref, w2_ref,
                b2_ref, out_ref):
    i = pl.program_id(0)
    ae = ae_ref[pl.ds(i * A_CHUNK, A_CHUNK), :]
    w1 = w1_ref[...]
    p = _mm_t(ae, w1[:, :EMB])
    rt = _mm_t(re_ref[...], w1[:, EMB:2 * EMB])
    u = lax.dot_general(bw_ref[...], w1[:, 2 * EMB:], (((0,), (1,)), ((), ())),
                        preferred_element_type=jnp.float32)
    v = lax.dot_general(bb_ref[...], w1[:, 2 * EMB:], (((0,), (1,)), ((), ())),
                        preferred_element_type=jnp.float32)
    bc = lax.broadcasted_iota(jnp.int32, (BC_STRIDE, 1), 0).astype(jnp.float32)
    bcu = lax.dot_general(bc, u, (((1,), (0,)), ((), ())),
                          preferred_element_type=jnp.float32)
    q = (rt[:, None, :] + bcu[None, :, :]
         + (v + b1_ref[...])[None, None, :])
    pre = p[:, None, None, :] + q[None, :, :, :]
    pre = pre.reshape(ROWS_PER_STEP, HID)
    h = pre * jax.nn.sigmoid(pre)
    out_ref[...] = _mm_t(h, w2_ref[...]) + b2_ref[...][None, :]


def _build_table(atom_emb, res_emb, bond_W, bond_b, W1, b1, W2, b2):
    return pl.pallas_call(
        _table_body,
        grid=(GRID_A,),
        in_specs=[
            pl.BlockSpec((N_ELEM, EMB), lambda i: (0, 0)),
            pl.BlockSpec((N_RES, EMB), lambda i: (0, 0)),
            pl.BlockSpec((EMB, 1), lambda i: (0, 0)),
            pl.BlockSpec((EMB,), lambda i: (0,)),
            pl.BlockSpec((HID, HID), lambda i: (0, 0)),
            pl.BlockSpec((HID,), lambda i: (0,)),
            pl.BlockSpec((HID, HID), lambda i: (0, 0)),
            pl.BlockSpec((HID,), lambda i: (0,)),
        ],
        out_specs=pl.BlockSpec((ROWS_PER_STEP, HID), lambda i: (i, 0)),
        out_shape=jax.ShapeDtypeStruct((N_TABLE, HID), jnp.float32),
    )(atom_emb, res_emb, bond_W, bond_b, W1, b1, W2, b2)


@functools.lru_cache(maxsize=1)
def _make_sc_idx():
    info = plsc.get_sparse_core_info()
    nc = info.num_cores
    mesh = plsc.VectorSubcoreMesh(core_axis_name="c", subcore_axis_name="s")

    @functools.partial(
        pl.kernel,
        mesh=mesh,
        out_type=jax.ShapeDtypeStruct((NW, N_CHUNKS, CHUNK), jnp.int32),
        scratch_types=[
            pltpu.VMEM((B_PER_W,), jnp.int32),
            pltpu.VMEM((B_PER_W,), jnp.int32),
            pltpu.VMEM((B_PER_W,), jnp.int32),
            pltpu.VMEM((N_CHUNKS, CHUNK), jnp.int32),
        ],
    )
    def idx_kernel(at_hbm, rt_hbm, bc_hbm, idx_hbm, a_v, r_v, b_v, idx_v):
        wid = lax.axis_index("s") * nc + lax.axis_index("c")
        base = wid * B_PER_W
        is_last_w0 = wid == NW - 1
        n_mine = N_ATOMS - (NW - 1) * B_PER_W

        @pl.when(is_last_w0)
        def _():
            pltpu.sync_copy(at_hbm.at[pl.ds(base, n_mine)],
                            a_v.at[pl.ds(0, n_mine)])
            pltpu.sync_copy(rt_hbm.at[pl.ds(base, n_mine)],
                            r_v.at[pl.ds(0, n_mine)])
            pltpu.sync_copy(bc_hbm.at[pl.ds(base, n_mine)],
                            b_v.at[pl.ds(0, n_mine)])

        @pl.when(jnp.logical_not(is_last_w0))
        def _():
            pltpu.sync_copy(at_hbm.at[pl.ds(base, B_PER_W)], a_v)
            pltpu.sync_copy(rt_hbm.at[pl.ds(base, B_PER_W)], r_v)
            pltpu.sync_copy(bc_hbm.at[pl.ds(base, B_PER_W)], b_v)

        def idx_body(j, carry):
            s = pl.ds(j * LANES, LANES)
            vals = (a_v[s] * (N_RES * BC_STRIDE) + r_v[s] * BC_STRIDE
                    + b_v[s])
            idx_v[j // (CHUNK // LANES),
                  pl.ds((j % (CHUNK // LANES)) * LANES, LANES)] = vals
            return carry
        n_idx = jnp.where(is_last_w0, n_mine // LANES, B_PER_W // LANES)
        lax.fori_loop(0, n_idx, idx_body, 0)

        @pl.when(is_last_w0)
        def _():
            zeros = jnp.zeros((LANES,), jnp.int32)
            for j in range((CHUNK - PART) // LANES):
                idx_v[LAST_FULL_LOCAL, pl.ds(PART + j * LANES, LANES)] = zeros

        pltpu.sync_copy(idx_v, idx_hbm.at[wid])

    return idx_kernel


@functools.lru_cache(maxsize=1)
def _make_sc_gather():
    info = plsc.get_sparse_core_info()
    nc = info.num_cores
    mesh = plsc.VectorSubcoreMesh(core_axis_name="c", subcore_axis_name="s")

    @functools.partial(
        pl.kernel,
        mesh=mesh,
        out_type=jax.ShapeDtypeStruct((N_ATOMS, HID), jnp.float32),
        scratch_types=(
            [pltpu.VMEM((N_CHUNKS, CHUNK), jnp.int32)]
            + [pltpu.VMEM((CHUNK, HID), jnp.float32)] * SLOTS
            + [pltpu.SemaphoreType.DMA] * (2 * SLOTS)
        ),
    )
    def gather(table_hbm, idx_hbm, out_hbm, idx_v, *scratch):
        bufs = scratch[:SLOTS]
        gsem = scratch[SLOTS:2 * SLOTS]
        wsem = scratch[2 * SLOTS:]
        wid = lax.axis_index("s") * nc + lax.axis_index("c")
        base = wid * B_PER_W
        is_last = wid == NW - 1
        not_last = jnp.logical_not(is_last)
        pltpu.sync_copy(idx_hbm.at[wid], idx_v)

        def sg(g):
            s = g % SLOTS
            pltpu.async_copy(table_hbm.at[idx_v.at[g]], bufs[s], gsem[s])

        def wg(g):
            s = g % SLOTS
            pltpu.make_async_copy(
                table_hbm.at[idx_v.at[g]], bufs[s], gsem[s]).wait()

        def sw(g):
            s = g % SLOTS
            pltpu.async_copy(
                bufs[s], out_hbm.at[pl.ds(base + g * CHUNK, CHUNK)], wsem[s])

        def ww(g):
            s = g % SLOTS
            pltpu.make_async_copy(
                bufs[s], out_hbm.at[pl.ds(base + g * CHUNK, CHUNK)],
                wsem[s]).wait()

        def sw_part(g):
            s = g % SLOTS
            pltpu.async_copy(
                bufs[s].at[pl.ds(0, PART)],
                out_hbm.at[pl.ds(base + g * CHUNK, PART)], wsem[s])

        def ww_part(g):
            s = g % SLOTS
            pltpu.make_async_copy(
                bufs[s].at[pl.ds(0, PART)],
                out_hbm.at[pl.ds(base + g * CHUNK, PART)], wsem[s]).wait()

        for g in range(N_CHUNKS):
            if g <= LAST_FULL_LOCAL:
                if g >= SLOTS:
                    ww(g - SLOTS)
                sg(g)
            else:
                @pl.when(not_last)
                def _(g=g):
                    ww(g - SLOTS)
                    sg(g)
            d = g - 2
            if d < 0:
                continue
            if d < LAST_FULL_LOCAL:
                wg(d)
                sw(d)
            elif d == LAST_FULL_LOCAL:
                wg(d)

                @pl.when(not_last)
                def _(d=d):
                    sw(d)

                @pl.when(is_last)
                def _(d=d):
                    sw_part(d)
            else:
                @pl.when(not_last)
                def _(d=d):
                    wg(d)
                    sw(d)

        @pl.when(not_last)
        def _():
            for d in range(N_CHUNKS - 2, N_CHUNKS):
                wg(d)
                sw(d)
            for w in range(N_CHUNKS - SLOTS, N_CHUNKS):
                ww(w)

        @pl.when(is_last)
        def _():
            for w in range(LAST_FULL_LOCAL - SLOTS + 1, LAST_FULL_LOCAL):
                ww(w)
            ww_part(LAST_FULL_LOCAL)

    return gather


def kernel(atom_type, residue_type, bond_count, atom_emb, res_emb, bond_W,
           bond_b, W1, b1, W2, b2):
    idx = _make_sc_idx()(atom_type, residue_type, bond_count)
    table = _build_table(atom_emb, res_emb, bond_W, bond_b, W1, b1, W2, b2)
    return _make_sc_gather()(table, idx)


# speedup vs baseline: 1.0289x; 1.0289x over previous
import functools

import jax
import jax.numpy as jnp
from jax import lax
from jax.experimental import pallas as pl
from jax.experimental.pallas import tpu as pltpu
from jax.experimental.pallas import tpu_sc as plsc

N_ELEM = 119
N_RES = 21
EMB = 128
HID = 384

A_CHUNK = 17
GRID_A = 7
BC_STRIDE = 8
ROWS_PER_STEP = A_CHUNK * N_RES * BC_STRIDE
N_TABLE = GRID_A * ROWS_PER_STEP

NW = 32
B_PER_W = 3200
CHUNK = 128
N_CHUNKS = B_PER_W // CHUNK
LANES = 16

N_ATOMS = 100000
LAST_FULL_LOCAL = N_ATOMS // CHUNK - (NW - 1) * N_CHUNKS
PART = N_ATOMS % CHUNK
assert LAST_FULL_LOCAL % 2 == 0 and PART % 8 == 0


def _mm_t(x, w):
    return lax.dot_general(x, w, (((1,), (1,)), ((), ())),
                           preferred_element_type=jnp.float32)


def _table_body(ae_ref, re_ref, bw_ref, bb_ref, w1_ref, b1_ref, w2_ref,
                b2_ref, out_ref):
    i = pl.program_id(0)
    ae = ae_ref[pl.ds(i * A_CHUNK, A_CHUNK), :]
    w1 = w1_ref[...]
    p = _mm_t(ae, w1[:, :EMB])
    rt = _mm_t(re_ref[...], w1[:, EMB:2 * EMB])
    u = lax.dot_general(bw_ref[...], w1[:, 2 * EMB:], (((0,), (1,)), ((), ())),
                        preferred_element_type=jnp.float32)
    v = lax.dot_general(bb_ref[...], w1[:, 2 * EMB:], (((0,), (1,)), ((), ())),
                        preferred_element_type=jnp.float32)
    bc = lax.broadcasted_iota(jnp.int32, (BC_STRIDE, 1), 0).astype(jnp.float32)
    bcu = lax.dot_general(bc, u, (((1,), (0,)), ((), ())),
                          preferred_element_type=jnp.float32)
    q = (rt[:, None, :] + bcu[None, :, :]
         + (v + b1_ref[...])[None, None, :])
    pre = p[:, None, None, :] + q[None, :, :, :]
    pre = pre.reshape(ROWS_PER_STEP, HID)
    h = pre * jax.nn.sigmoid(pre)
    out_ref[...] = _mm_t(h, w2_ref[...]) + b2_ref[...][None, :]


def _build_table(atom_emb, res_emb, bond_W, bond_b, W1, b1, W2, b2):
    return pl.pallas_call(
        _table_body,
        grid=(GRID_A,),
        in_specs=[
            pl.BlockSpec((N_ELEM, EMB), lambda i: (0, 0)),
            pl.BlockSpec((N_RES, EMB), lambda i: (0, 0)),
            pl.BlockSpec((EMB, 1), lambda i: (0, 0)),
            pl.BlockSpec((EMB,), lambda i: (0,)),
            pl.BlockSpec((HID, HID), lambda i: (0, 0)),
            pl.BlockSpec((HID,), lambda i: (0,)),
            pl.BlockSpec((HID, HID), lambda i: (0, 0)),
            pl.BlockSpec((HID,), lambda i: (0,)),
        ],
        out_specs=pl.BlockSpec((ROWS_PER_STEP, HID), lambda i: (i, 0)),
        out_shape=jax.ShapeDtypeStruct((N_TABLE, HID), jnp.float32),
    )(atom_emb, res_emb, bond_W, bond_b, W1, b1, W2, b2)


@functools.lru_cache(maxsize=1)
def _make_sc_idx():
    info = plsc.get_sparse_core_info()
    nc = info.num_cores
    mesh = plsc.VectorSubcoreMesh(core_axis_name="c", subcore_axis_name="s")

    @functools.partial(
        pl.kernel,
        mesh=mesh,
        out_type=jax.ShapeDtypeStruct((NW, N_CHUNKS, CHUNK), jnp.int32),
        scratch_types=[
            pltpu.VMEM((B_PER_W,), jnp.int32),
            pltpu.VMEM((B_PER_W,), jnp.int32),
            pltpu.VMEM((B_PER_W,), jnp.int32),
            pltpu.VMEM((N_CHUNKS, CHUNK), jnp.int32),
        ],
    )
    def idx_kernel(at_hbm, rt_hbm, bc_hbm, idx_hbm, a_v, r_v, b_v, idx_v):
        wid = lax.axis_index("s") * nc + lax.axis_index("c")
        base = wid * B_PER_W
        is_last_w0 = wid == NW - 1
        n_mine = N_ATOMS - (NW - 1) * B_PER_W

        @pl.when(is_last_w0)
        def _():
            pltpu.sync_copy(at_hbm.at[pl.ds(base, n_mine)],
                            a_v.at[pl.ds(0, n_mine)])
            pltpu.sync_copy(rt_hbm.at[pl.ds(base, n_mine)],
                            r_v.at[pl.ds(0, n_mine)])
            pltpu.sync_copy(bc_hbm.at[pl.ds(base, n_mine)],
                            b_v.at[pl.ds(0, n_mine)])

        @pl.when(jnp.logical_not(is_last_w0))
        def _():
            pltpu.sync_copy(at_hbm.at[pl.ds(base, B_PER_W)], a_v)
            pltpu.sync_copy(rt_hbm.at[pl.ds(base, B_PER_W)], r_v)
            pltpu.sync_copy(bc_hbm.at[pl.ds(base, B_PER_W)], b_v)

        def idx_body(j, carry):
            s = pl.ds(j * LANES, LANES)
            vals = (a_v[s] * (N_RES * BC_STRIDE) + r_v[s] * BC_STRIDE
                    + b_v[s])
            idx_v[j // (CHUNK // LANES),
                  pl.ds((j % (CHUNK // LANES)) * LANES, LANES)] = vals
            return carry
        n_idx = jnp.where(is_last_w0, n_mine // LANES, B_PER_W // LANES)
        lax.fori_loop(0, n_idx, idx_body, 0)

        @pl.when(is_last_w0)
        def _():
            zeros = jnp.zeros((LANES,), jnp.int32)
            for j in range((CHUNK - PART) // LANES):
                idx_v[LAST_FULL_LOCAL, pl.ds(PART + j * LANES, LANES)] = zeros

        pltpu.sync_copy(idx_v, idx_hbm.at[wid])

    return idx_kernel


@functools.lru_cache(maxsize=1)
def _make_sc_gather():
    info = plsc.get_sparse_core_info()
    nc = info.num_cores
    mesh = plsc.VectorSubcoreMesh(core_axis_name="c", subcore_axis_name="s")

    @functools.partial(
        pl.kernel,
        mesh=mesh,
        out_type=jax.ShapeDtypeStruct((N_ATOMS, HID), jnp.float32),
        scratch_types=[
            pltpu.VMEM((N_CHUNKS, CHUNK), jnp.int32),
            pltpu.VMEM((CHUNK, HID), jnp.float32),
            pltpu.VMEM((CHUNK, HID), jnp.float32),
            pltpu.SemaphoreType.DMA,
            pltpu.SemaphoreType.DMA,
            pltpu.SemaphoreType.DMA,
            pltpu.SemaphoreType.DMA,
        ],
    )
    def gather(table_hbm, idx_hbm, out_hbm,
               idx_v, buf0, buf1, semg0, semg1, semw0, semw1):
        wid = lax.axis_index("s") * nc + lax.axis_index("c")
        base = wid * B_PER_W
        pltpu.sync_copy(idx_hbm.at[wid], idx_v)

        def start_gather(g, buf, sem):
            pltpu.async_copy(table_hbm.at[idx_v.at[g]], buf, sem)

        def wait_gather(g, buf, sem):
            pltpu.make_async_copy(table_hbm.at[idx_v.at[g]], buf, sem).wait()

        def start_write(g, buf, sem):
            pltpu.async_copy(buf, out_hbm.at[pl.ds(base + g * CHUNK, CHUNK)],
                             sem)

        def wait_write(g, buf, sem):
            pltpu.make_async_copy(
                buf, out_hbm.at[pl.ds(base + g * CHUNK, CHUNK)], sem).wait()

        start_gather(0, buf0, semg0)

        def pair_body(t, carry):
            e, o = 2 * t, 2 * t + 1
            @pl.when(t > 0)
            def _():
                wait_write(e - 1, buf1, semw1)
            start_gather(o, buf1, semg1)
            wait_gather(e, buf0, semg0)
            start_write(e, buf0, semw0)
            wait_write(e, buf0, semw0)
            start_gather(o + 1, buf0, semg0)
            wait_gather(o, buf1, semg1)
            start_write(o, buf1, semw1)
            return carry
        is_last_w = wid == NW - 1
        npairs = jnp.where(is_last_w, LAST_FULL_LOCAL // 2,
                           (N_CHUNKS - 1) // 2)
        lax.fori_loop(0, npairs, pair_body, 0)

        @pl.when(jnp.logical_not(is_last_w))
        def _():
            last = N_CHUNKS - 1
            wait_gather(last, buf0, semg0)
            start_write(last, buf0, semw0)
            wait_write(last - 1, buf1, semw1)
            wait_write(last, buf0, semw0)

        @pl.when(is_last_w)
        def _():
            last = LAST_FULL_LOCAL
            wait_gather(last, buf0, semg0)
            pltpu.async_copy(
                buf0.at[pl.ds(0, PART)],
                out_hbm.at[pl.ds(base + last * CHUNK, PART)], semw0)
            wait_write(last - 1, buf1, semw1)
            pltpu.make_async_copy(
                buf0.at[pl.ds(0, PART)],
                out_hbm.at[pl.ds(base + last * CHUNK, PART)], semw0).wait()

    return gather


def kernel(atom_type, residue_type, bond_count, atom_emb, res_emb, bond_W,
           bond_b, W1, b1, W2, b2):
    idx = _make_sc_idx()(atom_type, residue_type, bond_count)
    table = _build_table(atom_emb, res_emb, bond_W, bond_b, W1, b1, W2, b2)
    return _make_sc_gather()(table, idx)
